# baseline (speedup 1.0000x reference)
import functools

import jax
import jax.numpy as jnp
from jax import lax
from jax.experimental import pallas as pl
from jax.experimental.pallas import tpu as pltpu
from jax.experimental.pallas import tpu_sc as plsc

_VOCAB = 1000000
_D = 32
_NC = 2
_NS = 16
_NW = _NC * _NS
_NI = 4096
_NJ = 200
_NJ0 = _NJ // 8
_NI0 = _NI // 128
_NSB = _NJ0 * _NI0
_SB_PER_W = _NSB // _NW


_STORE_BYTES = 8 * 4 * 8 * 128 * 4


def _gather_body(idx_hbm, table_hbm, out_hbm, idx_v, rows_v, trans_v,
                 gsem0, gsem1, ssem):
    wid = lax.axis_index("s") * _NC + lax.axis_index("c")
    gsem = (gsem0, gsem1)
    lane = lax.iota(jnp.int32, 16)

    def block_ids(n):
        sb = wid * _SB_PER_W + n
        return sb // _NI0, sb % _NI0

    def start_block(n, b):
        j0, i0 = block_ids(n)
        pltpu.sync_copy(idx_hbm.at[j0, i0], idx_v.at[b])
        for js in range(8):
            pltpu.async_copy(
                table_hbm.at[idx_v.at[b, js]], rows_v.at[b, js], gsem[b])

    def finish_block(n, b, wait_stores=True):
        j0, i0 = block_ids(n)
        for js in range(8):
            pltpu.make_async_copy(
                table_hbm.at[idx_v.at[b, js]], rows_v.at[b, js],
                gsem[b]).wait()
        if wait_stores:
            for js in range(8):
                pltpu.make_async_copy(
                    trans_v.at[js], out_hbm.at[8 * j0 + js, :, i0],
                    ssem).wait()

        @pl.loop(0, 8)
        def tr_g(g):
            is_vec = g * 16 + lane
            off = g * 16
            for js in range(8):
                for k in range(_D):
                    vals = plsc.load_gather(
                        rows_v.at[b, js],
                        [is_vec, jnp.full((16,), k, jnp.int32)])
                    trans_v[js, k // 8, k % 8, pl.ds(off, 16)] = vals

        for js in range(8):
            pltpu.async_copy(
                trans_v.at[js], out_hbm.at[8 * j0 + js, :, i0], ssem)

    start_block(0, 0)
    start_block(1, 1)
    finish_block(0, 0, wait_stores=False)
    start_block(2, 0)
    finish_block(1, 1)

    @pl.loop(2, _SB_PER_W - 1, step=2)
    def pair(g):
        start_block(g + 1, 1)
        finish_block(g, 0)
        start_block(g + 2, 0)
        finish_block(g + 1, 1)

    finish_block(_SB_PER_W - 1, 0)

    j0, i0 = block_ids(_SB_PER_W - 1)
    for js in range(8):
        pltpu.make_async_copy(
            trans_v.at[js], out_hbm.at[8 * j0 + js, :, i0], ssem).wait()


@jax.jit
def _lookup(idx4, table):
    mesh = plsc.VectorSubcoreMesh(
        core_axis_name="c", subcore_axis_name="s",
        num_cores=_NC, num_subcores=_NS,
    )
    f = pl.kernel(
        _gather_body,
        out_type=jax.ShapeDtypeStruct((_NJ, _D // 8, _NI0, 8, 128),
                                      jnp.float32),
        mesh=mesh,
        scratch_types=[
            pltpu.VMEM((2, 8, 128), jnp.int32),
            pltpu.VMEM((2, 8, 128, _D), jnp.float32),
            pltpu.VMEM((8, _D // 8, 8, 128), jnp.float32),
            pltpu.SemaphoreType.DMA,
            pltpu.SemaphoreType.DMA,
            pltpu.SemaphoreType.DMA,
        ],
        compiler_params=pltpu.CompilerParams(
            use_tc_tiling_on_sc=False, needs_layout_passes=False),
    )
    return f(idx4, table)


def kernel(indices, table):
    idx4 = indices.astype(jnp.int32).reshape(32, 128, 25, 8).transpose(2, 0, 3, 1)
    out5 = _lookup(idx4, table)
    return out5.transpose(2, 4, 0, 1, 3).reshape(_NI, _NJ, _D)


# speedup vs baseline: 1.1232x; 1.1232x over previous
import functools

import jax
import jax.numpy as jnp
from jax import lax
from jax.experimental import pallas as pl
from jax.experimental.pallas import tpu as pltpu
from jax.experimental.pallas import tpu_sc as plsc

_VOCAB = 1000000
_D = 32
_NC = 2
_NS = 16
_NW = _NC * _NS
_NI = 4096
_NJ = 200
_NJ0 = _NJ // 8
_NI0 = _NI // 128
_NSB = _NJ0 * _NI0
_SB_PER_W = _NSB // _NW


_STORE_BYTES = 8 * 4 * 8 * 128 * 4


def _gather_body(idx_hbm, table_hbm, out_hbm, idx_v, rows_v, pad_v, trans_v,
                 gsem0, gsem1, ssem):
    wid = lax.axis_index("s") * _NC + lax.axis_index("c")
    gsem = (gsem0, gsem1)
    lane = lax.iota(jnp.int32, 16)

    def block_ids(n):
        sb = wid * _SB_PER_W + n
        return sb // _NI0, sb % _NI0

    def start_block(n, b):
        j0, i0 = block_ids(n)
        pltpu.sync_copy(idx_hbm.at[j0, i0], idx_v.at[b])
        for js in range(8):
            pltpu.async_copy(
                table_hbm.at[idx_v.at[b, js]], rows_v.at[b, js], gsem[b])

    def finish_block(n, b, wait_stores=True):
        j0, i0 = block_ids(n)
        for js in range(8):
            pltpu.make_async_copy(
                table_hbm.at[idx_v.at[b, js]], rows_v.at[b, js],
                gsem[b]).wait()
        if wait_stores:
            for js in range(8):
                pltpu.make_async_copy(
                    trans_v.at[js], out_hbm.at[8 * j0 + js, :, i0],
                    ssem).wait()

        @pl.loop(0, 8)
        def per_js(js):
            @pl.loop(0, 128, step=16)
            def pad(s0):
                for t in range(16):
                    s = s0 + t
                    pad_v[s, pl.ds(0, 16)] = rows_v[b, js, s, pl.ds(0, 16)]
                    pad_v[s, pl.ds(16, 16)] = rows_v[b, js, s, pl.ds(16, 16)]

            @pl.loop(0, _D, step=8)
            def trk(k0):
                for kk in range(8):
                    k = k0 + kk
                    for g in range(8):
                        vals = plsc.load_gather(
                            pad_v,
                            [g * 16 + lane, jnp.full((16,), k, jnp.int32)])
                        trans_v[js, k0 // 8, kk, pl.ds(g * 16, 16)] = vals

        for js in range(8):
            pltpu.async_copy(
                trans_v.at[js], out_hbm.at[8 * j0 + js, :, i0], ssem)

    start_block(0, 0)
    start_block(1, 1)
    finish_block(0, 0, wait_stores=False)
    start_block(2, 0)
    finish_block(1, 1)

    @pl.loop(2, _SB_PER_W - 1, step=2)
    def pair(g):
        start_block(g + 1, 1)
        finish_block(g, 0)
        start_block(g + 2, 0)
        finish_block(g + 1, 1)

    finish_block(_SB_PER_W - 1, 0)

    j0, i0 = block_ids(_SB_PER_W - 1)
    for js in range(8):
        pltpu.make_async_copy(
            trans_v.at[js], out_hbm.at[8 * j0 + js, :, i0], ssem).wait()


@jax.jit
def _lookup(idx4, table):
    mesh = plsc.VectorSubcoreMesh(
        core_axis_name="c", subcore_axis_name="s",
        num_cores=_NC, num_subcores=_NS,
    )
    f = pl.kernel(
        _gather_body,
        out_type=jax.ShapeDtypeStruct((_NJ, _D // 8, _NI0, 8, 128),
                                      jnp.float32),
        mesh=mesh,
        scratch_types=[
            pltpu.VMEM((2, 8, 128), jnp.int32),
            pltpu.VMEM((2, 8, 128, _D), jnp.float32),
            pltpu.VMEM((128, 33), jnp.float32),
            pltpu.VMEM((8, _D // 8, 8, 128), jnp.float32),
            pltpu.SemaphoreType.DMA,
            pltpu.SemaphoreType.DMA,
            pltpu.SemaphoreType.DMA,
        ],
        compiler_params=pltpu.CompilerParams(
            use_tc_tiling_on_sc=False, needs_layout_passes=False),
    )
    return f(idx4, table)


def kernel(indices, table):
    idx4 = indices.astype(jnp.int32).reshape(32, 128, 25, 8).transpose(2, 0, 3, 1)
    out5 = _lookup(idx4, table)
    return out5.transpose(2, 4, 0, 1, 3).reshape(_NI, _NJ, _D)


# speedup vs baseline: 1.8240x; 1.6239x over previous
import functools

import jax
import jax.numpy as jnp
from jax import lax
from jax.experimental import pallas as pl
from jax.experimental.pallas import tpu as pltpu
from jax.experimental.pallas import tpu_sc as plsc

_VOCAB = 1000000
_D = 32
_NC = 2
_NS = 16
_NW = _NC * _NS
_NI = 4096
_NJ = 200
_NJ0 = _NJ // 8
_NI0 = _NI // 128
_NSB = _NJ0 * _NI0
_SB_PER_W = _NSB // _NW


_STORE_BYTES = 8 * 4 * 8 * 128 * 4


def _gather_body(idx_hbm, table_hbm, out_hbm, idx_v, rows_v, pad_v, trans_v,
                 gsem0, gsem1, ssem):
    wid = lax.axis_index("s") * _NC + lax.axis_index("c")
    gsem = (gsem0, gsem1)
    lane = lax.iota(jnp.int32, 16)

    def block_ids(n):
        sb = wid * _SB_PER_W + n
        return sb // _NI0, sb % _NI0

    def start_block(n, b):
        j0, i0 = block_ids(n)
        pltpu.sync_copy(idx_hbm.at[j0, i0], idx_v.at[b])
        for js in range(8):
            pltpu.async_copy(
                table_hbm.at[idx_v.at[b, js]], rows_v.at[b, js], gsem[b])

    def finish_block(n, b, wait_stores=True):
        j0, i0 = block_ids(n)
        for js in range(8):
            pltpu.make_async_copy(
                table_hbm.at[idx_v.at[b, js]], rows_v.at[b, js],
                gsem[b]).wait()
        if wait_stores:
            for js in range(8):
                pltpu.make_async_copy(
                    trans_v.at[js], out_hbm.at[8 * j0 + js, :, i0],
                    ssem).wait()

        @pl.loop(0, 8)
        def per_js(js):
            rows_js = rows_v.at[b, js]

            @plsc.parallel_loop(0, 128, step=16, unroll=2)
            def pad(s0):
                for t in range(16):
                    s = s0 + t
                    pad_v[s, pl.ds(0, 16)] = rows_js[s, pl.ds(0, 16)]
                    pad_v[s, pl.ds(16, 16)] = rows_js[s, pl.ds(16, 16)]

            @plsc.parallel_loop(0, _D, unroll=2)
            def trk(k):
                for g in range(8):
                    vals = plsc.load_gather(
                        pad_v,
                        [g * 16 + lane, jnp.full((16,), k, jnp.int32)])
                    trans_v[js, k // 8, k % 8, pl.ds(g * 16, 16)] = vals

        for js in range(8):
            pltpu.async_copy(
                trans_v.at[js], out_hbm.at[8 * j0 + js, :, i0], ssem)

    start_block(0, 0)
    start_block(1, 1)
    finish_block(0, 0, wait_stores=False)
    start_block(2, 0)
    finish_block(1, 1)

    @pl.loop(2, _SB_PER_W - 1, step=2)
    def pair(g):
        start_block(g + 1, 1)
        finish_block(g, 0)
        start_block(g + 2, 0)
        finish_block(g + 1, 1)

    finish_block(_SB_PER_W - 1, 0)

    j0, i0 = block_ids(_SB_PER_W - 1)
    for js in range(8):
        pltpu.make_async_copy(
            trans_v.at[js], out_hbm.at[8 * j0 + js, :, i0], ssem).wait()


@jax.jit
def _lookup(idx4, table):
    mesh = plsc.VectorSubcoreMesh(
        core_axis_name="c", subcore_axis_name="s",
        num_cores=_NC, num_subcores=_NS,
    )
    f = pl.kernel(
        _gather_body,
        out_type=jax.ShapeDtypeStruct((_NJ, _D // 8, _NI0, 8, 128),
                                      jnp.float32),
        mesh=mesh,
        scratch_types=[
            pltpu.VMEM((2, 8, 128), jnp.int32),
            pltpu.VMEM((2, 8, 128, _D), jnp.float32),
            pltpu.VMEM((128, 33), jnp.float32),
            pltpu.VMEM((8, _D // 8, 8, 128), jnp.float32),
            pltpu.SemaphoreType.DMA,
            pltpu.SemaphoreType.DMA,
            pltpu.SemaphoreType.DMA,
        ],
        compiler_params=pltpu.CompilerParams(
            use_tc_tiling_on_sc=False, needs_layout_passes=False),
    )
    return f(idx4, table)


def kernel(indices, table):
    idx4 = indices.astype(jnp.int32).reshape(32, 128, 25, 8).transpose(2, 0, 3, 1)
    out5 = _lookup(idx4, table)
    return out5.transpose(2, 4, 0, 1, 3).reshape(_NI, _NJ, _D)


# speedup vs baseline: 1.9533x; 1.0709x over previous
import jax
import jax.numpy as jnp
from jax import lax
from jax.experimental import pallas as pl
from jax.experimental.pallas import tpu as pltpu
from jax.experimental.pallas import tpu_sc as plsc

_VOCAB = 1000000
_D = 32
_NC = 2
_NS = 16
_NW = _NC * _NS
_NI = 4096
_NJ = 200
_NJ0 = _NJ // 8
_NI0 = _NI // 128
_NSB = _NJ0 * _NI0
_SB_PER_W = _NSB // _NW

_NT = _VOCAB // 128
_T_PER_W = _NT // _NW


def _detile_body(tt_hbm, tail_hbm, trm_hbm, slab_v, pad_v, drm_v,
                 lsem0, lsem1, ssem):
    wid = lax.axis_index("s") * _NC + lax.axis_index("c")
    lsem = (lsem0, lsem1)
    lane = lax.iota(jnp.int32, 16)

    def r0_of(t):
        return (wid + _NW * t) * 128

    def q0_of(t):
        return (wid + _NW * t) * 32

    def start_block(t, b):
        pltpu.async_copy(
            tt_hbm.at[:, pl.ds(r0_of(t), 128)], slab_v.at[b], lsem[b])

    def transpose_to(b, width):
        @plsc.parallel_loop(0, 32, unroll=2)
        def padc(c):
            for h in range(width // 16):
                pad_v[c, pl.ds(h * 16, 16)] = slab_v[b, c, pl.ds(h * 16, 16)]

        @plsc.parallel_loop(0, width, unroll=2)
        def trr(r):
            for cg in range(2):
                vals = plsc.load_gather(
                    pad_v,
                    [cg * 16 + lane, jnp.full((16,), r, jnp.int32)])
                drm_v[b, r // 4, pl.ds((r % 4) * _D + cg * 16, 16)] = vals

    def finish_block(t, b, wait_store=True):
        r0 = r0_of(t)
        pltpu.make_async_copy(
            tt_hbm.at[:, pl.ds(r0, 128)], slab_v.at[b], lsem[b]).wait()
        if wait_store:
            pltpu.make_async_copy(
                drm_v.at[b], trm_hbm.at[pl.ds(q0_of(t), 32)], ssem).wait()
        transpose_to(b, 128)
        pltpu.async_copy(drm_v.at[b], trm_hbm.at[pl.ds(q0_of(t), 32)], ssem)

    start_block(0, 0)
    start_block(1, 1)
    finish_block(0, 0, wait_store=False)
    start_block(2, 0)
    finish_block(1, 1, wait_store=False)
    start_block(3, 1)

    @pl.loop(2, _T_PER_W - 2, step=2)
    def pair(g):
        finish_block(g, 0)
        start_block(g + 2, 0)
        finish_block(g + 1, 1)
        start_block(g + 3, 1)

    finish_block(_T_PER_W - 2, 0)
    finish_block(_T_PER_W - 1, 1)
    pltpu.make_async_copy(
        drm_v.at[0], trm_hbm.at[pl.ds(q0_of(_T_PER_W - 2), 32)], ssem).wait()
    pltpu.make_async_copy(
        drm_v.at[1], trm_hbm.at[pl.ds(q0_of(_T_PER_W - 1), 32)], ssem).wait()

    @pl.when(wid == 4)
    def tail():
        pltpu.sync_copy(tail_hbm, drm_v.at[0, pl.ds(0, 16)])
        pltpu.sync_copy(drm_v.at[0, pl.ds(0, 16)],
                        trm_hbm.at[pl.ds(_NT * _D, 16)])


def _fused_body(idx_hbm, table_hbm, out_hbm, idx_v, rows_v, pad_v, trans_v,
                gsem0, gsem1, ssem):
    wid = lax.axis_index("s") * _NC + lax.axis_index("c")
    gsem = (gsem0, gsem1)
    lane = lax.iota(jnp.int32, 16)

    def block_ids(n):
        sb = wid * _SB_PER_W + n
        return sb // _NI0, sb % _NI0

    def start_block(n, b):
        j0, i0 = block_ids(n)
        pltpu.sync_copy(idx_hbm.at[j0, i0], idx_v.at[b])
        for js in range(8):
            pltpu.async_copy(
                table_hbm.at[idx_v.at[b, js]], rows_v.at[b, js], gsem[b])

    def finish_block(n, b, wait_stores=True):
        j0, i0 = block_ids(n)
        for js in range(8):
            pltpu.make_async_copy(
                table_hbm.at[idx_v.at[b, js]], rows_v.at[b, js],
                gsem[b]).wait()
        if wait_stores:
            for js in range(8):
                pltpu.make_async_copy(
                    trans_v.at[js], out_hbm.at[8 * j0 + js, :, i0],
                    ssem).wait()

        @pl.loop(0, 8)
        def per_js(js):
            rows_js = rows_v.at[b, js]

            @plsc.parallel_loop(0, 128, step=16, unroll=2)
            def pad(s0):
                for t in range(16):
                    s = s0 + t
                    pad_v[s, pl.ds(0, 16)] = rows_js[s, pl.ds(0, 16)]
                    pad_v[s, pl.ds(16, 16)] = rows_js[s, pl.ds(16, 16)]

            @plsc.parallel_loop(0, _D, unroll=2)
            def trk(k):
                for g in range(8):
                    vals = plsc.load_gather(
                        pad_v,
                        [g * 16 + lane, jnp.full((16,), k, jnp.int32)])
                    trans_v[js, k // 8, k % 8, pl.ds(g * 16, 16)] = vals

        for js in range(8):
            pltpu.async_copy(
                trans_v.at[js], out_hbm.at[8 * j0 + js, :, i0], ssem)

    start_block(0, 0)
    start_block(1, 1)
    finish_block(0, 0, wait_stores=False)
    start_block(2, 0)
    finish_block(1, 1)

    @pl.loop(2, _SB_PER_W - 1, step=2)
    def pair(g):
        start_block(g + 1, 1)
        finish_block(g, 0)
        start_block(g + 2, 0)
        finish_block(g + 1, 1)

    finish_block(_SB_PER_W - 1, 0)

    j0, i0 = block_ids(_SB_PER_W - 1)
    for js in range(8):
        pltpu.make_async_copy(
            trans_v.at[js], out_hbm.at[8 * j0 + js, :, i0], ssem).wait()


def _mesh():
    return plsc.VectorSubcoreMesh(
        core_axis_name="c", subcore_axis_name="s",
        num_cores=_NC, num_subcores=_NS,
    )


@jax.jit
def _lookup(idx4, table_t, table_tail):
    table_rm = pl.kernel(
        _detile_body,
        out_type=jax.ShapeDtypeStruct((_VOCAB * _D // 128, 128), jnp.float32),
        mesh=_mesh(),
        scratch_types=[
            pltpu.VMEM((2, _D, 128), jnp.float32),
            pltpu.VMEM((_D, 129), jnp.float32),
            pltpu.VMEM((2, 32, 128), jnp.float32),
            pltpu.SemaphoreType.DMA,
            pltpu.SemaphoreType.DMA,
            pltpu.SemaphoreType.DMA,
        ],
        compiler_params=pltpu.CompilerParams(
            use_tc_tiling_on_sc=True, needs_layout_passes=False),
    )(table_t, table_tail)

    out5 = pl.kernel(
        _fused_body,
        out_type=jax.ShapeDtypeStruct((_NJ, _D // 8, _NI0, 8, 128),
                                      jnp.float32),
        mesh=_mesh(),
        scratch_types=[
            pltpu.VMEM((2, 8, 128), jnp.int32),
            pltpu.VMEM((2, 8, 128, _D), jnp.float32),
            pltpu.VMEM((128, 33), jnp.float32),
            pltpu.VMEM((8, _D // 8, 8, 128), jnp.float32),
            pltpu.SemaphoreType.DMA,
            pltpu.SemaphoreType.DMA,
            pltpu.SemaphoreType.DMA,
        ],
        compiler_params=pltpu.CompilerParams(
            use_tc_tiling_on_sc=False, needs_layout_passes=False),
    )(idx4, table_rm.reshape(_VOCAB, _D))
    return out5


def kernel(indices, table):
    idx4 = indices.astype(jnp.int32).reshape(32, 128, 25, 8).transpose(2, 0, 3, 1)
    out5 = _lookup(idx4, table.T, table[_NT * 128:].reshape(16, 128))
    return out5.transpose(2, 4, 0, 1, 3).reshape(_NI, _NJ, _D)
